# baseline (speedup 1.0000x reference)
import functools

import jax
import jax.numpy as jnp
from jax import lax
from jax.experimental import pallas as pl
from jax.experimental.pallas import tpu as pltpu
from jax.experimental.pallas import tpu_sc as plsc

N = 10000
E = 320000
NL = 128
NH = 64
NC = 2
NS = 16
NW = NC * NS
CH = 128
NCH = 79
EPAD = NW * NCH * CH
NPOOL = NW * NCH
FAST_CORE = 1
NCH_FAST = 113
NCH_SLOW = 2 * NCH - NCH_FAST
ACC_ROWS = 10240
DEGW = 16
BM = 2000
_F32 = jnp.float32


def _sc_mesh():
    return plsc.VectorSubcoreMesh(core_axis_name="c", subcore_axis_name="s")


_SC_PARAMS = pltpu.CompilerParams(use_tc_tiling_on_sc=False)


@functools.partial(
    pl.kernel,
    out_type=jax.ShapeDtypeStruct((NC, ACC_ROWS, DEGW), _F32),
    mesh=_sc_mesh(),
    scratch_types=[
        pltpu.VMEM((NCH, CH), jnp.int32),
        pltpu.VMEM((CH, DEGW), _F32),
        pltpu.VMEM((CH, DEGW), _F32),
        pltpu.VMEM((CH, DEGW), _F32),
        pltpu.VMEM_SHARED((ACC_ROWS, DEGW), _F32),
    ],
    compiler_params=_SC_PARAMS,
)
def _deg_kernel(dst_hbm, ones_hbm, zeros_hbm, out_hbm,
                dst_vm, ones_vm, zero_vm, buf_vm, acc):
    c = lax.axis_index("c")
    s = lax.axis_index("s")
    wid = c * NS + s
    pltpu.sync_copy(dst_hbm.at[pl.ds(wid * NCH, NCH)], dst_vm)
    pltpu.sync_copy(ones_hbm, ones_vm)
    pltpu.sync_copy(zeros_hbm, zero_vm)
    for z in range(ACC_ROWS // (NS * CH)):
        pltpu.sync_copy(zero_vm, acc.at[pl.ds(s * (ACC_ROWS // NS) + z * CH, CH)])
    plsc.subcore_barrier()

    def body(j, carry):
        pltpu.sync_copy(ones_vm, acc.at[dst_vm.at[j]], add=True)
        return carry

    lax.fori_loop(0, NCH, body, 0)
    plsc.subcore_barrier()
    for z in range(ACC_ROWS // (NS * CH)):
        r = s * (ACC_ROWS // NS) + z * CH
        pltpu.sync_copy(acc.at[pl.ds(r, CH)], buf_vm)
        pltpu.sync_copy(buf_vm, out_hbm.at[c].at[pl.ds(r, CH)])


@functools.partial(
    pl.kernel,
    out_type=jax.ShapeDtypeStruct((NC, ACC_ROWS, NH), _F32),
    mesh=_sc_mesh(),
    scratch_types=[
        pltpu.VMEM((NCH_FAST, CH), jnp.int32),
        pltpu.VMEM((NCH_FAST, CH), jnp.int32),
        pltpu.VMEM((6, CH, NH), _F32),
        pltpu.VMEM((CH, NH), _F32),
        pltpu.VMEM_SHARED((ACC_ROWS, NH), _F32),
        pltpu.SemaphoreType.DMA((6,)),
        pltpu.SemaphoreType.DMA((6,)),
    ],
    compiler_params=_SC_PARAMS,
)
def _scatter_kernel(y_hbm, src_hbm, dst_hbm, zeros_hbm, out_hbm,
                    src_vm, dst_vm, rows_vm, zero_vm, acc, gsem, ssem):
    c = lax.axis_index("c")
    s = lax.axis_index("s")
    cnt = jnp.where(c == FAST_CORE, NCH_FAST, NCH_SLOW)
    base = jnp.where(c == FAST_CORE, NS * NCH_SLOW + s * NCH_FAST,
                     s * NCH_SLOW)
    pltpu.sync_copy(src_hbm.at[pl.ds(base, NCH_FAST)], src_vm)
    pltpu.sync_copy(dst_hbm.at[pl.ds(base, NCH_FAST)], dst_vm)
    pltpu.sync_copy(zeros_hbm, zero_vm)
    for z in range(ACC_ROWS // (NS * CH)):
        pltpu.sync_copy(zero_vm, acc.at[pl.ds(s * (ACC_ROWS // NS) + z * CH, CH)])
    plsc.subcore_barrier()

    def _gather(j):
        b = lax.rem(j, 6)
        pltpu.async_copy(y_hbm.at[src_vm.at[j]], rows_vm.at[b], gsem.at[b])

    def _scatter(j):
        b = lax.rem(j, 6)
        return pltpu.make_async_copy(rows_vm.at[b], acc.at[dst_vm.at[j]],
                                     ssem.at[b])

    for j in range(3):
        _gather(j)

    def body(j, carry):
        b = lax.rem(j, 6)
        pltpu.make_async_copy(y_hbm.at[src_vm.at[j]], rows_vm.at[b],
                              gsem.at[b]).wait()
        _scatter(j).start(add=True)

        @pl.when(j >= 3)
        def _():
            _scatter(j - 3).wait()

        @pl.when(j + 3 < cnt)
        def _():
            _gather(j + 3)

        return carry

    lax.fori_loop(0, cnt, body, 0)

    def drain(j, carry):
        _scatter(j).wait()
        return carry

    lax.fori_loop(cnt - 3, cnt, drain, 0)
    plsc.subcore_barrier()
    for z in range(ACC_ROWS // (NS * CH)):
        r = s * (ACC_ROWS // NS) + z * CH
        pltpu.sync_copy(acc.at[pl.ds(r, CH)], zero_vm)
        pltpu.sync_copy(zero_vm, out_hbm.at[c].at[pl.ds(r, CH)])


def _deg_stats(da_ref, db_ref):
    deg = da_ref[:, 0:1] + db_ref[:, 0:1] + 1.0
    return lax.rsqrt(deg), 1.0 / deg


def _tc_a_body(lg_ref, w_ref, da_ref, db_ref, xs_ref, self1_ref):
    dis, inv = _deg_stats(da_ref, db_ref)
    x1 = jnp.dot(lg_ref[...], w_ref[...], preferred_element_type=_F32)
    xs_ref[...] = x1 * dis
    self1_ref[...] = x1 * inv


def _tc_b_body(sa_ref, sb_ref, da_ref, db_ref, self1_ref, b1_ref,
               hs_ref, self2_ref):
    dis, inv = _deg_stats(da_ref, db_ref)
    h = dis * (sa_ref[...] + sb_ref[...]) + self1_ref[...] + b1_ref[...]
    h = jnp.maximum(h, 0.0)
    hs_ref[...] = h * dis
    self2_ref[...] = h * inv


def _tc_c_body(sa_ref, sb_ref, da_ref, db_ref, self2_ref, w_ref, b2_ref,
               out_ref):
    dis, inv = _deg_stats(da_ref, db_ref)
    p = dis * (sa_ref[...] + sb_ref[...]) + self2_ref[...]
    out_ref[...] = jnp.dot(p, w_ref[...], preferred_element_type=_F32) + b2_ref[...]


def _rows(shape_minor):
    return pl.BlockSpec((BM, shape_minor), lambda i: (i, 0))


def _whole(r, c_):
    return pl.BlockSpec((r, c_), lambda i: (0, 0))


_GRID = (N // BM,)


def _tc_a(logits, w1, dega, degb):
    return pl.pallas_call(
        _tc_a_body,
        grid=_GRID,
        in_specs=[_rows(NL), _whole(NL, NH), _rows(DEGW), _rows(DEGW)],
        out_specs=[_rows(NH), _rows(NH)],
        out_shape=[jax.ShapeDtypeStruct((N, NH), _F32)] * 2,
    )(logits, w1, dega, degb)


def _tc_b(s1a, s1b, dega, degb, self1, b1):
    return pl.pallas_call(
        _tc_b_body,
        grid=_GRID,
        in_specs=[_rows(NH), _rows(NH), _rows(DEGW), _rows(DEGW),
                  _rows(NH), _whole(1, NH)],
        out_specs=[_rows(NH), _rows(NH)],
        out_shape=[jax.ShapeDtypeStruct((N, NH), _F32)] * 2,
    )(s1a, s1b, dega, degb, self1, b1)


def _tc_c(s2a, s2b, dega, degb, self2, w2, b2):
    return pl.pallas_call(
        _tc_c_body,
        grid=_GRID,
        in_specs=[_rows(NH), _rows(NH), _rows(DEGW), _rows(DEGW),
                  _rows(NH), _whole(NH, NL), _whole(1, NL)],
        out_specs=_rows(NL),
        out_shape=jax.ShapeDtypeStruct((N, NL), _F32),
    )(s2a, s2b, dega, degb, self2, w2, b2)


def kernel(logits, edge_index, W1, b1, W2, b2):
    assert logits.shape == (N, NL) and edge_index.shape == (2, E)
    src = edge_index[0].astype(jnp.int32)
    dst = edge_index[1].astype(jnp.int32)
    pad = EPAD - E
    src_p = jnp.concatenate([src, jnp.zeros((pad,), jnp.int32)]).reshape(NPOOL, CH)
    dst_p = jnp.concatenate([dst, jnp.full((pad,), N, jnp.int32)]).reshape(NPOOL, CH)

    ones16 = jnp.ones((CH, DEGW), _F32)
    zeros16 = jnp.zeros((CH, DEGW), _F32)
    zeros64 = jnp.zeros((CH, NH), _F32)

    degp = _deg_kernel(dst_p, ones16, zeros16)
    dega, degb = degp[0, :N], degp[1, :N]

    xs, self1 = _tc_a(logits, W1, dega, degb)
    s1 = _scatter_kernel(xs, src_p, dst_p, zeros64)
    hs, self2 = _tc_b(s1[0, :N], s1[1, :N], dega, degb, self1,
                      b1.reshape(1, NH))
    s2 = _scatter_kernel(hs, src_p, dst_p, zeros64)
    return _tc_c(s2[0, :N], s2[1, :N], dega, degb, self2, W2,
                 b2.reshape(1, NL))


# speedup vs baseline: 1.0523x; 1.0523x over previous
import functools

import jax
import jax.numpy as jnp
from jax import lax
from jax.experimental import pallas as pl
from jax.experimental.pallas import tpu as pltpu
from jax.experimental.pallas import tpu_sc as plsc

N = 10000
E = 320000
NL = 128
NH = 64
NC = 2
NS = 16
NW = NC * NS
CH = 128
NCH = 79
EPAD = NW * NCH * CH
NPOOL = NW * NCH
FAST_CORE = 1
NCH_FAST = 79
NCH_SLOW = 2 * NCH - NCH_FAST
ACC_ROWS = 10240
DEGW = 16
BM = 2000
_F32 = jnp.float32


def _sc_mesh():
    return plsc.VectorSubcoreMesh(core_axis_name="c", subcore_axis_name="s")


_SC_PARAMS = pltpu.CompilerParams(use_tc_tiling_on_sc=False)


@functools.partial(
    pl.kernel,
    out_type=jax.ShapeDtypeStruct((NC, ACC_ROWS, DEGW), _F32),
    mesh=_sc_mesh(),
    scratch_types=[
        pltpu.VMEM((NCH, CH), jnp.int32),
        pltpu.VMEM((CH, DEGW), _F32),
        pltpu.VMEM((CH, DEGW), _F32),
        pltpu.VMEM((CH, DEGW), _F32),
        pltpu.VMEM_SHARED((ACC_ROWS, DEGW), _F32),
    ],
    compiler_params=_SC_PARAMS,
)
def _deg_kernel(dst_hbm, ones_hbm, zeros_hbm, out_hbm,
                dst_vm, ones_vm, zero_vm, buf_vm, acc):
    c = lax.axis_index("c")
    s = lax.axis_index("s")
    wid = c * NS + s
    pltpu.sync_copy(dst_hbm.at[pl.ds(wid * NCH, NCH)], dst_vm)
    pltpu.sync_copy(ones_hbm, ones_vm)
    pltpu.sync_copy(zeros_hbm, zero_vm)
    for z in range(ACC_ROWS // (NS * CH)):
        pltpu.sync_copy(zero_vm, acc.at[pl.ds(s * (ACC_ROWS // NS) + z * CH, CH)])
    plsc.subcore_barrier()

    def body(j, carry):
        pltpu.sync_copy(ones_vm, acc.at[dst_vm.at[j]], add=True)
        return carry

    lax.fori_loop(0, NCH, body, 0)
    plsc.subcore_barrier()
    for z in range(ACC_ROWS // (NS * CH)):
        r = s * (ACC_ROWS // NS) + z * CH
        pltpu.sync_copy(acc.at[pl.ds(r, CH)], buf_vm)
        pltpu.sync_copy(buf_vm, out_hbm.at[c].at[pl.ds(r, CH)])


@functools.partial(
    pl.kernel,
    out_type=jax.ShapeDtypeStruct((NC, ACC_ROWS, NH), _F32),
    mesh=_sc_mesh(),
    scratch_types=[
        pltpu.VMEM((NCH_FAST, CH), jnp.int32),
        pltpu.VMEM((NCH_FAST, CH), jnp.int32),
        pltpu.VMEM((6, CH, NH), _F32),
        pltpu.VMEM((CH, NH), _F32),
        pltpu.VMEM_SHARED((ACC_ROWS, NH), _F32),
        pltpu.SemaphoreType.DMA((6,)),
        pltpu.SemaphoreType.DMA((6,)),
    ],
    compiler_params=_SC_PARAMS,
)
def _scatter_kernel(y_hbm, src_hbm, dst_hbm, zeros_hbm, out_hbm,
                    src_vm, dst_vm, rows_vm, zero_vm, acc, gsem, ssem):
    c = lax.axis_index("c")
    s = lax.axis_index("s")
    cnt = jnp.where(c == FAST_CORE, NCH_FAST, NCH_SLOW)
    base = jnp.where(c == FAST_CORE, NS * NCH_SLOW + s * NCH_FAST,
                     s * NCH_SLOW)
    pltpu.sync_copy(src_hbm.at[pl.ds(base, NCH_FAST)], src_vm)
    pltpu.sync_copy(dst_hbm.at[pl.ds(base, NCH_FAST)], dst_vm)
    pltpu.sync_copy(zeros_hbm, zero_vm)
    for z in range(ACC_ROWS // (NS * CH)):
        pltpu.sync_copy(zero_vm, acc.at[pl.ds(s * (ACC_ROWS // NS) + z * CH, CH)])
    plsc.subcore_barrier()

    def _gather(j):
        b = lax.rem(j, 6)
        pltpu.async_copy(y_hbm.at[src_vm.at[j]], rows_vm.at[b], gsem.at[b])

    def _scatter(j):
        b = lax.rem(j, 6)
        return pltpu.make_async_copy(rows_vm.at[b], acc.at[dst_vm.at[j]],
                                     ssem.at[b])

    for j in range(3):
        _gather(j)

    def body(j, carry):
        b = lax.rem(j, 6)
        pltpu.make_async_copy(y_hbm.at[src_vm.at[j]], rows_vm.at[b],
                              gsem.at[b]).wait()
        _scatter(j).start(add=True)

        @pl.when(j >= 3)
        def _():
            _scatter(j - 3).wait()

        @pl.when(j + 3 < cnt)
        def _():
            _gather(j + 3)

        return carry

    lax.fori_loop(0, cnt, body, 0)

    def drain(j, carry):
        _scatter(j).wait()
        return carry

    lax.fori_loop(cnt - 3, cnt, drain, 0)
    plsc.subcore_barrier()
    for z in range(ACC_ROWS // (NS * CH)):
        r = s * (ACC_ROWS // NS) + z * CH
        pltpu.sync_copy(acc.at[pl.ds(r, CH)], zero_vm)
        pltpu.sync_copy(zero_vm, out_hbm.at[c].at[pl.ds(r, CH)])


def _deg_stats(da_ref, db_ref):
    deg = da_ref[:, 0:1] + db_ref[:, 0:1] + 1.0
    return lax.rsqrt(deg), 1.0 / deg


def _tc_a_body(lg_ref, w_ref, da_ref, db_ref, xs_ref, self1_ref):
    dis, inv = _deg_stats(da_ref, db_ref)
    x1 = jnp.dot(lg_ref[...], w_ref[...], preferred_element_type=_F32)
    xs_ref[...] = x1 * dis
    self1_ref[...] = x1 * inv


def _tc_b_body(sa_ref, sb_ref, da_ref, db_ref, self1_ref, b1_ref,
               hs_ref, self2_ref):
    dis, inv = _deg_stats(da_ref, db_ref)
    h = dis * (sa_ref[...] + sb_ref[...]) + self1_ref[...] + b1_ref[...]
    h = jnp.maximum(h, 0.0)
    hs_ref[...] = h * dis
    self2_ref[...] = h * inv


def _tc_c_body(sa_ref, sb_ref, da_ref, db_ref, self2_ref, w_ref, b2_ref,
               out_ref):
    dis, inv = _deg_stats(da_ref, db_ref)
    p = dis * (sa_ref[...] + sb_ref[...]) + self2_ref[...]
    out_ref[...] = jnp.dot(p, w_ref[...], preferred_element_type=_F32) + b2_ref[...]


def _rows(shape_minor):
    return pl.BlockSpec((BM, shape_minor), lambda i: (i, 0))


def _whole(r, c_):
    return pl.BlockSpec((r, c_), lambda i: (0, 0))


_GRID = (N // BM,)


def _tc_a(logits, w1, dega, degb):
    return pl.pallas_call(
        _tc_a_body,
        grid=_GRID,
        in_specs=[_rows(NL), _whole(NL, NH), _rows(DEGW), _rows(DEGW)],
        out_specs=[_rows(NH), _rows(NH)],
        out_shape=[jax.ShapeDtypeStruct((N, NH), _F32)] * 2,
    )(logits, w1, dega, degb)


def _tc_b(s1a, s1b, dega, degb, self1, b1):
    return pl.pallas_call(
        _tc_b_body,
        grid=_GRID,
        in_specs=[_rows(NH), _rows(NH), _rows(DEGW), _rows(DEGW),
                  _rows(NH), _whole(1, NH)],
        out_specs=[_rows(NH), _rows(NH)],
        out_shape=[jax.ShapeDtypeStruct((N, NH), _F32)] * 2,
    )(s1a, s1b, dega, degb, self1, b1)


def _tc_c(s2a, s2b, dega, degb, self2, w2, b2):
    return pl.pallas_call(
        _tc_c_body,
        grid=_GRID,
        in_specs=[_rows(NH), _rows(NH), _rows(DEGW), _rows(DEGW),
                  _rows(NH), _whole(NH, NL), _whole(1, NL)],
        out_specs=_rows(NL),
        out_shape=jax.ShapeDtypeStruct((N, NL), _F32),
    )(s2a, s2b, dega, degb, self2, w2, b2)


def kernel(logits, edge_index, W1, b1, W2, b2):
    assert logits.shape == (N, NL) and edge_index.shape == (2, E)
    src = edge_index[0].astype(jnp.int32)
    dst = edge_index[1].astype(jnp.int32)
    pad = EPAD - E
    pad_dst = N + jnp.arange(pad, dtype=jnp.int32) % (ACC_ROWS - N)
    src_p = jnp.concatenate([src, jnp.zeros((pad,), jnp.int32)]).reshape(NPOOL, CH)
    dst_p = jnp.concatenate([dst, pad_dst]).reshape(NPOOL, CH)

    ones16 = jnp.ones((CH, DEGW), _F32)
    zeros16 = jnp.zeros((CH, DEGW), _F32)
    zeros64 = jnp.zeros((CH, NH), _F32)

    degp = _deg_kernel(dst_p, ones16, zeros16)
    dega, degb = degp[0, :N], degp[1, :N]

    xs, self1 = _tc_a(logits, W1, dega, degb)
    s1 = _scatter_kernel(xs, src_p, dst_p, zeros64)
    hs, self2 = _tc_b(s1[0, :N], s1[1, :N], dega, degb, self1,
                      b1.reshape(1, NH))
    s2 = _scatter_kernel(hs, src_p, dst_p, zeros64)
    return _tc_c(s2[0, :N], s2[1, :N], dega, degb, self2, W2,
                 b2.reshape(1, NL))


# speedup vs baseline: 1.7329x; 1.6467x over previous
import functools

import jax
import jax.numpy as jnp
from jax import lax
from jax.experimental import pallas as pl
from jax.experimental.pallas import tpu as pltpu
from jax.experimental.pallas import tpu_sc as plsc

N = 10000
E = 320000
NL = 128
NH = 64
NC = 2
NS = 16
NW = NC * NS
CH = 128
NCH = 79
EPAD = NW * NCH * CH
NPOOL = NW * NCH
FAST_CORE = 1
NCH_FAST = 79
NCH_SLOW = 2 * NCH - NCH_FAST
ACC_ROWS = 10240
DEGW = 16
BM = 2000
_F32 = jnp.float32


def _sc_mesh():
    return plsc.VectorSubcoreMesh(core_axis_name="c", subcore_axis_name="s")


_SC_PARAMS = pltpu.CompilerParams(use_tc_tiling_on_sc=False)


@functools.partial(
    pl.kernel,
    out_type=jax.ShapeDtypeStruct((NC, ACC_ROWS, DEGW), _F32),
    mesh=_sc_mesh(),
    scratch_types=[
        pltpu.VMEM((NCH, CH), jnp.int32),
        pltpu.VMEM((CH, DEGW), _F32),
        pltpu.VMEM((CH, DEGW), _F32),
        pltpu.VMEM((CH, DEGW), _F32),
        pltpu.VMEM_SHARED((ACC_ROWS, DEGW), _F32),
    ],
    compiler_params=_SC_PARAMS,
)
def _deg_kernel(dst_hbm, ones_hbm, zeros_hbm, out_hbm,
                dst_vm, ones_vm, zero_vm, buf_vm, acc):
    c = lax.axis_index("c")
    s = lax.axis_index("s")
    wid = c * NS + s
    pltpu.sync_copy(dst_hbm.at[pl.ds(wid * NCH, NCH)], dst_vm)
    pltpu.sync_copy(ones_hbm, ones_vm)
    pltpu.sync_copy(zeros_hbm, zero_vm)
    for z in range(ACC_ROWS // (NS * CH)):
        pltpu.sync_copy(zero_vm, acc.at[pl.ds(s * (ACC_ROWS // NS) + z * CH, CH)])
    plsc.subcore_barrier()

    def body(j, carry):
        pltpu.sync_copy(ones_vm, acc.at[dst_vm.at[j]], add=True)
        return carry

    lax.fori_loop(0, NCH, body, 0)
    plsc.subcore_barrier()
    for z in range(ACC_ROWS // (NS * CH)):
        r = s * (ACC_ROWS // NS) + z * CH
        pltpu.sync_copy(acc.at[pl.ds(r, CH)], buf_vm)
        pltpu.sync_copy(buf_vm, out_hbm.at[c].at[pl.ds(r, CH)])


@functools.partial(
    pl.kernel,
    out_type=jax.ShapeDtypeStruct((NC, ACC_ROWS, NH), _F32),
    mesh=_sc_mesh(),
    scratch_types=[
        pltpu.VMEM((NCH_FAST, CH), jnp.int32),
        pltpu.VMEM((NCH_FAST, CH), jnp.int32),
        pltpu.VMEM((6, CH, NH), _F32),
        pltpu.VMEM((CH, NH), _F32),
        pltpu.VMEM_SHARED((ACC_ROWS, NH), _F32),
        pltpu.SemaphoreType.DMA((6,)),
        pltpu.SemaphoreType.DMA((6,)),
    ],
    compiler_params=_SC_PARAMS,
)
def _scatter_kernel(y_hbm, src_hbm, dst_hbm, zeros_hbm, out_hbm,
                    src_vm, dst_vm, rows_vm, zero_vm, acc, gsem, ssem):
    c = lax.axis_index("c")
    s = lax.axis_index("s")
    cnt = jnp.where(c == FAST_CORE, NCH_FAST, NCH_SLOW)
    base = jnp.where(c == FAST_CORE, NS * NCH_SLOW + s * NCH_FAST,
                     s * NCH_SLOW)
    pltpu.sync_copy(src_hbm.at[pl.ds(base, NCH_FAST)], src_vm)
    pltpu.sync_copy(dst_hbm.at[pl.ds(base, NCH_FAST)], dst_vm)
    pltpu.sync_copy(zeros_hbm, zero_vm)
    for z in range(ACC_ROWS // (NS * CH)):
        pltpu.sync_copy(zero_vm, acc.at[pl.ds(s * (ACC_ROWS // NS) + z * CH, CH)])
    plsc.subcore_barrier()

    def _gather(j):
        b = lax.rem(j, 6)
        pltpu.async_copy(y_hbm.at[src_vm.at[j]], rows_vm.at[b], gsem.at[b])

    def _scatter(j):
        b = lax.rem(j, 6)
        return pltpu.make_async_copy(rows_vm.at[b], acc.at[dst_vm.at[j]],
                                     ssem.at[b])

    for j in range(3):
        _gather(j)

    def body(j, carry):
        b = lax.rem(j, 6)
        pltpu.make_async_copy(y_hbm.at[src_vm.at[j]], rows_vm.at[b],
                              gsem.at[b]).wait()
        _scatter(j).start(add=True)

        @pl.when(j >= 3)
        def _():
            _scatter(j - 3).wait()

        @pl.when(j + 3 < cnt)
        def _():
            _gather(j + 3)

        return carry

    lax.fori_loop(0, cnt, body, 0)

    def drain(j, carry):
        _scatter(j).wait()
        return carry

    lax.fori_loop(cnt - 3, cnt, drain, 0)
    plsc.subcore_barrier()
    for z in range(ACC_ROWS // (NS * CH)):
        r = s * (ACC_ROWS // NS) + z * CH
        pltpu.sync_copy(acc.at[pl.ds(r, CH)], zero_vm)
        pltpu.sync_copy(zero_vm, out_hbm.at[c].at[pl.ds(r, CH)])


def _deg_stats(da_ref, db_ref):
    deg = da_ref[:, 0:1] + db_ref[:, 0:1] + 1.0
    return lax.rsqrt(deg), 1.0 / deg


def _tc_a_body(lg_ref, w_ref, da_ref, db_ref, xs_ref, self1_ref):
    dis, inv = _deg_stats(da_ref, db_ref)
    x1 = jnp.dot(lg_ref[...], w_ref[...], preferred_element_type=_F32)
    xs_ref[...] = x1 * dis
    self1_ref[...] = x1 * inv


def _tc_b_body(sa_ref, sb_ref, da_ref, db_ref, self1_ref, b1_ref,
               hs_ref, self2_ref):
    dis, inv = _deg_stats(da_ref, db_ref)
    h = dis * (sa_ref[...] + sb_ref[...]) + self1_ref[...] + b1_ref[...]
    h = jnp.maximum(h, 0.0)
    hs_ref[...] = h * dis
    self2_ref[...] = h * inv


def _tc_c_body(sa_ref, sb_ref, da_ref, db_ref, self2_ref, w_ref, b2_ref,
               out_ref):
    dis, inv = _deg_stats(da_ref, db_ref)
    p = dis * (sa_ref[...] + sb_ref[...]) + self2_ref[...]
    out_ref[...] = jnp.dot(p, w_ref[...], preferred_element_type=_F32) + b2_ref[...]


def _rows(shape_minor):
    return pl.BlockSpec((BM, shape_minor), lambda i: (i, 0))


def _whole(r, c_):
    return pl.BlockSpec((r, c_), lambda i: (0, 0))


_GRID = (N // BM,)


def _tc_a(logits, w1, dega, degb):
    return pl.pallas_call(
        _tc_a_body,
        grid=_GRID,
        in_specs=[_rows(NL), _whole(NL, NH), _rows(DEGW), _rows(DEGW)],
        out_specs=[_rows(NH), _rows(NH)],
        out_shape=[jax.ShapeDtypeStruct((N, NH), _F32)] * 2,
    )(logits, w1, dega, degb)


def _tc_b(s1a, s1b, dega, degb, self1, b1):
    return pl.pallas_call(
        _tc_b_body,
        grid=_GRID,
        in_specs=[_rows(NH), _rows(NH), _rows(DEGW), _rows(DEGW),
                  _rows(NH), _whole(1, NH)],
        out_specs=[_rows(NH), _rows(NH)],
        out_shape=[jax.ShapeDtypeStruct((N, NH), _F32)] * 2,
    )(s1a, s1b, dega, degb, self1, b1)


def _tc_c(s2a, s2b, dega, degb, self2, w2, b2):
    return pl.pallas_call(
        _tc_c_body,
        grid=_GRID,
        in_specs=[_rows(NH), _rows(NH), _rows(DEGW), _rows(DEGW),
                  _rows(NH), _whole(NH, NL), _whole(1, NL)],
        out_specs=_rows(NL),
        out_shape=jax.ShapeDtypeStruct((N, NL), _F32),
    )(s2a, s2b, dega, degb, self2, w2, b2)


def kernel(logits, edge_index, W1, b1, W2, b2):
    assert logits.shape == (N, NL) and edge_index.shape == (2, E)
    src = edge_index[0].astype(jnp.int32)
    dst = edge_index[1].astype(jnp.int32)
    pad = EPAD - E
    pad_dst = N + jnp.arange(pad, dtype=jnp.int32) % (ACC_ROWS - N)
    pad_src = jnp.arange(pad, dtype=jnp.int32) % N
    src_p = jnp.concatenate([src, pad_src]).reshape(NPOOL, CH)
    dst_p = jnp.concatenate([dst, pad_dst]).reshape(NPOOL, CH)

    ones16 = jnp.ones((CH, DEGW), _F32)
    zeros16 = jnp.zeros((CH, DEGW), _F32)
    zeros64 = jnp.zeros((CH, NH), _F32)

    degp = _deg_kernel(dst_p, ones16, zeros16)
    dega, degb = degp[0, :N], degp[1, :N]

    xs, self1 = _tc_a(logits, W1, dega, degb)
    s1 = _scatter_kernel(xs, src_p, dst_p, zeros64)
    hs, self2 = _tc_b(s1[0, :N], s1[1, :N], dega, degb, self1,
                      b1.reshape(1, NH))
    s2 = _scatter_kernel(hs, src_p, dst_p, zeros64)
    return _tc_c(s2[0, :N], s2[1, :N], dega, degb, self2, W2,
                 b2.reshape(1, NL))


# speedup vs baseline: 1.8963x; 1.0943x over previous
import functools

import jax
import jax.numpy as jnp
from jax import lax
from jax.experimental import pallas as pl
from jax.experimental.pallas import tpu as pltpu
from jax.experimental.pallas import tpu_sc as plsc

N = 10000
E = 320000
NL = 128
NH = 64
NC = 2
NS = 16
NW = NC * NS
CH = 128
NCH = 79
EPAD = NW * NCH * CH
NPOOL = NW * NCH
FAST_CORE = 1
NCH_FAST = 79
NCH_SLOW = 2 * NCH - NCH_FAST
ACC_ROWS = 10240
DEGW = 16
BM = 2000
_F32 = jnp.float32


def _sc_mesh():
    return plsc.VectorSubcoreMesh(core_axis_name="c", subcore_axis_name="s")


_SC_PARAMS = pltpu.CompilerParams(use_tc_tiling_on_sc=False)


@functools.partial(
    pl.kernel,
    out_type=jax.ShapeDtypeStruct((NC, ACC_ROWS, DEGW), _F32),
    mesh=_sc_mesh(),
    scratch_types=[
        pltpu.VMEM((NCH, CH), jnp.int32),
        pltpu.VMEM((CH, DEGW), _F32),
        pltpu.VMEM((CH, DEGW), _F32),
        pltpu.VMEM((CH, DEGW), _F32),
        pltpu.VMEM_SHARED((ACC_ROWS, DEGW), _F32),
    ],
    compiler_params=_SC_PARAMS,
)
def _deg_kernel(dst_hbm, ones_hbm, zeros_hbm, out_hbm,
                dst_vm, ones_vm, zero_vm, buf_vm, acc):
    c = lax.axis_index("c")
    s = lax.axis_index("s")
    wid = c * NS + s
    pltpu.sync_copy(dst_hbm.at[pl.ds(wid * NCH, NCH)], dst_vm)
    pltpu.sync_copy(ones_hbm, ones_vm)
    pltpu.sync_copy(zeros_hbm, zero_vm)
    for z in range(ACC_ROWS // (NS * CH)):
        pltpu.sync_copy(zero_vm, acc.at[pl.ds(s * (ACC_ROWS // NS) + z * CH, CH)])
    plsc.subcore_barrier()

    def body(j, carry):
        pltpu.sync_copy(ones_vm, acc.at[dst_vm.at[j]], add=True)
        return carry

    lax.fori_loop(0, NCH, body, 0)
    plsc.subcore_barrier()
    for z in range(ACC_ROWS // (NS * CH)):
        r = s * (ACC_ROWS // NS) + z * CH
        pltpu.sync_copy(acc.at[pl.ds(r, CH)], buf_vm)
        pltpu.sync_copy(buf_vm, out_hbm.at[c].at[pl.ds(r, CH)])


@functools.partial(
    pl.kernel,
    out_type=jax.ShapeDtypeStruct((NC, ACC_ROWS, NH), _F32),
    mesh=_sc_mesh(),
    scratch_types=[
        pltpu.VMEM((NCH_FAST, CH), jnp.int32),
        pltpu.VMEM((NCH_FAST, CH), jnp.int32),
        pltpu.VMEM((6, CH, NH), _F32),
        pltpu.VMEM((CH, NH), _F32),
        pltpu.VMEM_SHARED((ACC_ROWS, NH), _F32),
        pltpu.SemaphoreType.DMA((6,)),
        pltpu.SemaphoreType.DMA((6,)),
    ],
    compiler_params=_SC_PARAMS,
)
def _scatter_kernel(y_hbm, src_hbm, dst_hbm, zeros_hbm, out_hbm,
                    src_vm, dst_vm, rows_vm, zero_vm, acc, gsem, ssem):
    c = lax.axis_index("c")
    s = lax.axis_index("s")
    cnt = jnp.where(c == FAST_CORE, NCH_FAST, NCH_SLOW)
    base = jnp.where(c == FAST_CORE, NS * NCH_SLOW + s * NCH_FAST,
                     s * NCH_SLOW)
    pltpu.sync_copy(src_hbm.at[pl.ds(base, NCH_FAST)], src_vm)
    pltpu.sync_copy(dst_hbm.at[pl.ds(base, NCH_FAST)], dst_vm)
    pltpu.sync_copy(zeros_hbm, zero_vm)
    for z in range(ACC_ROWS // (NS * CH)):
        pltpu.sync_copy(zero_vm, acc.at[pl.ds(s * (ACC_ROWS // NS) + z * CH, CH)])
    plsc.subcore_barrier()

    def _gather(j):
        b = lax.rem(j, 6)
        pltpu.async_copy(y_hbm.at[src_vm.at[j]], rows_vm.at[b], gsem.at[b])

    def _scatter(j):
        b = lax.rem(j, 6)
        return pltpu.make_async_copy(rows_vm.at[b], acc.at[dst_vm.at[j]],
                                     ssem.at[b])

    for j in range(3):
        _gather(j)

    def body(j, carry):
        b = lax.rem(j, 6)
        pltpu.make_async_copy(y_hbm.at[src_vm.at[j]], rows_vm.at[b],
                              gsem.at[b]).wait()
        _scatter(j).start(add=True)

        @pl.when(j >= 3)
        def _():
            _scatter(j - 3).wait()

        @pl.when(j + 3 < cnt)
        def _():
            _gather(j + 3)

        return carry

    lax.fori_loop(0, cnt, body, 0)

    def drain(j, carry):
        _scatter(j).wait()
        return carry

    lax.fori_loop(cnt - 3, cnt, drain, 0)
    plsc.subcore_barrier()
    for z in range(ACC_ROWS // (NS * CH)):
        r = s * (ACC_ROWS // NS) + z * CH
        pltpu.sync_copy(acc.at[pl.ds(r, CH)], zero_vm)
        pltpu.sync_copy(zero_vm, out_hbm.at[c].at[pl.ds(r, CH)])


def _deg_stats(da_ref, db_ref):
    deg = da_ref[0, :, 0:1] + db_ref[0, :, 0:1] + 1.0
    return lax.rsqrt(deg), 1.0 / deg


def _tc_mm_body(lg_ref, w_ref, x1_ref):
    x1_ref[...] = jnp.dot(lg_ref[...], w_ref[...], preferred_element_type=_F32)


def _tc_scale_body(x1_ref, da_ref, db_ref, xs_ref, self1_ref):
    dis, inv = _deg_stats(da_ref, db_ref)
    x1 = x1_ref[...]
    xs_ref[...] = x1 * dis
    self1_ref[...] = x1 * inv


def _tc_b_body(sa_ref, sb_ref, da_ref, db_ref, self1_ref, b1_ref,
               hs_ref, self2_ref):
    dis, inv = _deg_stats(da_ref, db_ref)
    h = dis * (sa_ref[0] + sb_ref[0]) + self1_ref[...] + b1_ref[...]
    h = jnp.maximum(h, 0.0)
    hs_ref[...] = h * dis
    self2_ref[...] = h * inv


def _tc_c_body(sa_ref, sb_ref, da_ref, db_ref, self2_ref, w_ref, b2_ref,
               out_ref):
    dis, inv = _deg_stats(da_ref, db_ref)
    p = dis * (sa_ref[0] + sb_ref[0]) + self2_ref[...]
    out_ref[...] = jnp.dot(p, w_ref[...], preferred_element_type=_F32) + b2_ref[...]


def _rows(shape_minor):
    return pl.BlockSpec((BM, shape_minor), lambda i: (i, 0))


def _slab(shape_minor, k):
    return pl.BlockSpec((1, BM, shape_minor), lambda i, _k=k: (_k, i, 0))


def _whole(r, c_):
    return pl.BlockSpec((r, c_), lambda i: (0, 0))


_GRID = (N // BM,)


def _tc_mm(logits, w1):
    return pl.pallas_call(
        _tc_mm_body,
        grid=_GRID,
        in_specs=[_rows(NL), _whole(NL, NH)],
        out_specs=_rows(NH),
        out_shape=jax.ShapeDtypeStruct((N, NH), _F32),
    )(logits, w1)


def _tc_scale(x1, degp):
    return pl.pallas_call(
        _tc_scale_body,
        grid=_GRID,
        in_specs=[_rows(NH), _slab(DEGW, 0), _slab(DEGW, 1)],
        out_specs=[_rows(NH), _rows(NH)],
        out_shape=[jax.ShapeDtypeStruct((N, NH), _F32)] * 2,
    )(x1, degp, degp)


def _tc_b(s1, degp, self1, b1):
    return pl.pallas_call(
        _tc_b_body,
        grid=_GRID,
        in_specs=[_slab(NH, 0), _slab(NH, 1), _slab(DEGW, 0), _slab(DEGW, 1),
                  _rows(NH), _whole(1, NH)],
        out_specs=[_rows(NH), _rows(NH)],
        out_shape=[jax.ShapeDtypeStruct((N, NH), _F32)] * 2,
    )(s1, s1, degp, degp, self1, b1)


def _tc_c(s2, degp, self2, w2, b2):
    return pl.pallas_call(
        _tc_c_body,
        grid=_GRID,
        in_specs=[_slab(NH, 0), _slab(NH, 1), _slab(DEGW, 0), _slab(DEGW, 1),
                  _rows(NH), _whole(NH, NL), _whole(1, NL)],
        out_specs=_rows(NL),
        out_shape=jax.ShapeDtypeStruct((N, NL), _F32),
    )(s2, s2, degp, degp, self2, w2, b2)


def kernel(logits, edge_index, W1, b1, W2, b2):
    assert logits.shape == (N, NL) and edge_index.shape == (2, E)
    src = edge_index[0].astype(jnp.int32)
    dst = edge_index[1].astype(jnp.int32)
    pad = EPAD - E
    pad_dst = N + jnp.arange(pad, dtype=jnp.int32) % (ACC_ROWS - N)
    pad_src = jnp.arange(pad, dtype=jnp.int32) % N
    src_p = jnp.concatenate([src, pad_src]).reshape(NPOOL, CH)
    dst_p = jnp.concatenate([dst, pad_dst]).reshape(NPOOL, CH)

    ones16 = jnp.ones((CH, DEGW), _F32)
    zeros16 = jnp.zeros((CH, DEGW), _F32)
    zeros64 = jnp.zeros((CH, NH), _F32)

    degp = _deg_kernel(dst_p, ones16, zeros16)
    x1 = _tc_mm(logits, W1)
    xs, self1 = _tc_scale(x1, degp)
    s1 = _scatter_kernel(xs, src_p, dst_p, zeros64)
    hs, self2 = _tc_b(s1, degp, self1, b1.reshape(1, NH))
    s2 = _scatter_kernel(hs, src_p, dst_p, zeros64)
    return _tc_c(s2, degp, self2, W2, b2.reshape(1, NL))


# speedup vs baseline: 2.0155x; 1.0629x over previous
import functools

import jax
import jax.numpy as jnp
from jax import lax
from jax.experimental import pallas as pl
from jax.experimental.pallas import tpu as pltpu
from jax.experimental.pallas import tpu_sc as plsc

N = 10000
E = 320000
NL = 128
NH = 64
NC = 2
NS = 16
NW = NC * NS
CH = 128
NCH = 79
EPAD = NW * NCH * CH
NPOOL = NW * NCH
FAST_CORE = 1
NCH_FAST = 79
NCH_SLOW = 2 * NCH - NCH_FAST
ACC_ROWS = 10240
DEGW = 16
BM = 2000
_F32 = jnp.float32


def _sc_mesh():
    return plsc.VectorSubcoreMesh(core_axis_name="c", subcore_axis_name="s")


_SC_PARAMS = pltpu.CompilerParams(use_tc_tiling_on_sc=False)


@functools.partial(
    pl.kernel,
    out_type=jax.ShapeDtypeStruct((NC, ACC_ROWS, DEGW), _F32),
    mesh=_sc_mesh(),
    scratch_types=[
        pltpu.VMEM((NCH, CH), jnp.int32),
        pltpu.VMEM((CH, DEGW), _F32),
        pltpu.VMEM((CH, DEGW), _F32),
        pltpu.VMEM((CH, DEGW), _F32),
        pltpu.VMEM_SHARED((ACC_ROWS, DEGW), _F32),
    ],
    compiler_params=_SC_PARAMS,
)
def _deg_kernel(dst_hbm, ones_hbm, zeros_hbm, out_hbm,
                dst_vm, ones_vm, zero_vm, buf_vm, acc):
    c = lax.axis_index("c")
    s = lax.axis_index("s")
    wid = c * NS + s
    pltpu.sync_copy(dst_hbm.at[pl.ds(wid * NCH, NCH)], dst_vm)
    pltpu.sync_copy(ones_hbm, ones_vm)
    pltpu.sync_copy(zeros_hbm, zero_vm)
    for z in range(ACC_ROWS // (NS * CH)):
        pltpu.sync_copy(zero_vm, acc.at[pl.ds(s * (ACC_ROWS // NS) + z * CH, CH)])
    plsc.subcore_barrier()

    def body(j, carry):
        pltpu.sync_copy(ones_vm, acc.at[dst_vm.at[j]], add=True)
        return carry

    lax.fori_loop(0, NCH, body, 0)
    plsc.subcore_barrier()
    for z in range(ACC_ROWS // (NS * CH)):
        r = s * (ACC_ROWS // NS) + z * CH
        pltpu.sync_copy(acc.at[pl.ds(r, CH)], buf_vm)
        pltpu.sync_copy(buf_vm, out_hbm.at[c].at[pl.ds(r, CH)])


@functools.partial(
    pl.kernel,
    out_type=jax.ShapeDtypeStruct((NC, ACC_ROWS, NH), _F32),
    mesh=_sc_mesh(),
    scratch_types=[
        pltpu.VMEM((NCH_FAST, CH), jnp.int32),
        pltpu.VMEM((NCH_FAST, CH), jnp.int32),
        pltpu.VMEM((6, CH, NH), _F32),
        pltpu.VMEM((CH, NH), _F32),
        pltpu.VMEM_SHARED((ACC_ROWS, NH), _F32),
        pltpu.SemaphoreType.DMA((6,)),
        pltpu.SemaphoreType.DMA((6,)),
    ],
    compiler_params=_SC_PARAMS,
)
def _scatter_kernel(y_hbm, src_hbm, dst_hbm, zeros_hbm, out_hbm,
                    src_vm, dst_vm, rows_vm, zero_vm, acc, gsem, ssem):
    c = lax.axis_index("c")
    s = lax.axis_index("s")
    cnt = jnp.where(c == FAST_CORE, NCH_FAST, NCH_SLOW)
    base = jnp.where(c == FAST_CORE, NS * NCH_SLOW + s * NCH_FAST,
                     s * NCH_SLOW)
    pltpu.sync_copy(src_hbm.at[pl.ds(base, NCH_FAST)], src_vm)
    pltpu.sync_copy(dst_hbm.at[pl.ds(base, NCH_FAST)], dst_vm)
    pltpu.sync_copy(zeros_hbm, zero_vm)
    for z in range(ACC_ROWS // (NS * CH)):
        pltpu.sync_copy(zero_vm, acc.at[pl.ds(s * (ACC_ROWS // NS) + z * CH, CH)])
    plsc.subcore_barrier()

    def _gather(j):
        b = lax.rem(j, 6)
        pltpu.async_copy(y_hbm.at[src_vm.at[j]], rows_vm.at[b], gsem.at[b])

    def _scatter(j):
        b = lax.rem(j, 6)
        return pltpu.make_async_copy(rows_vm.at[b], acc.at[dst_vm.at[j]],
                                     ssem.at[b])

    for j in range(4):
        _gather(j)

    def body(j, carry):
        b = lax.rem(j, 6)
        pltpu.make_async_copy(y_hbm.at[src_vm.at[j]], rows_vm.at[b],
                              gsem.at[b]).wait()
        _scatter(j).start(add=True)

        @pl.when(j >= 2)
        def _():
            _scatter(j - 2).wait()

        @pl.when(j + 4 < cnt)
        def _():
            _gather(j + 4)

        return carry

    lax.fori_loop(0, cnt, body, 0)

    def drain(j, carry):
        _scatter(j).wait()
        return carry

    lax.fori_loop(cnt - 2, cnt, drain, 0)
    plsc.subcore_barrier()
    for z in range(ACC_ROWS // (NS * CH)):
        r = s * (ACC_ROWS // NS) + z * CH
        pltpu.sync_copy(acc.at[pl.ds(r, CH)], zero_vm)
        pltpu.sync_copy(zero_vm, out_hbm.at[c].at[pl.ds(r, CH)])


def _deg_stats(da_ref, db_ref):
    deg = da_ref[0, :, 0:1] + db_ref[0, :, 0:1] + 1.0
    return lax.rsqrt(deg), 1.0 / deg


def _tc_mm_body(lg_ref, w_ref, x1_ref):
    x1_ref[...] = jnp.dot(lg_ref[...], w_ref[...], preferred_element_type=_F32)


def _tc_scale_body(x1_ref, da_ref, db_ref, xs_ref, self1_ref):
    dis, inv = _deg_stats(da_ref, db_ref)
    x1 = x1_ref[...]
    xs_ref[...] = x1 * dis
    self1_ref[...] = x1 * inv


def _tc_b_body(sa_ref, sb_ref, da_ref, db_ref, self1_ref, b1_ref,
               hs_ref, self2_ref):
    dis, inv = _deg_stats(da_ref, db_ref)
    h = dis * (sa_ref[0] + sb_ref[0]) + self1_ref[...] + b1_ref[...]
    h = jnp.maximum(h, 0.0)
    hs_ref[...] = h * dis
    self2_ref[...] = h * inv


def _tc_c_body(sa_ref, sb_ref, da_ref, db_ref, self2_ref, w_ref, b2_ref,
               out_ref):
    dis, inv = _deg_stats(da_ref, db_ref)
    p = dis * (sa_ref[0] + sb_ref[0]) + self2_ref[...]
    out_ref[...] = jnp.dot(p, w_ref[...], preferred_element_type=_F32) + b2_ref[...]


def _rows(shape_minor):
    return pl.BlockSpec((BM, shape_minor), lambda i: (i, 0))


def _slab(shape_minor, k):
    return pl.BlockSpec((1, BM, shape_minor), lambda i, _k=k: (_k, i, 0))


def _whole(r, c_):
    return pl.BlockSpec((r, c_), lambda i: (0, 0))


_GRID = (N // BM,)


def _tc_mm(logits, w1):
    return pl.pallas_call(
        _tc_mm_body,
        grid=_GRID,
        in_specs=[_rows(NL), _whole(NL, NH)],
        out_specs=_rows(NH),
        out_shape=jax.ShapeDtypeStruct((N, NH), _F32),
    )(logits, w1)


def _tc_scale(x1, degp):
    return pl.pallas_call(
        _tc_scale_body,
        grid=_GRID,
        in_specs=[_rows(NH), _slab(DEGW, 0), _slab(DEGW, 1)],
        out_specs=[_rows(NH), _rows(NH)],
        out_shape=[jax.ShapeDtypeStruct((N, NH), _F32)] * 2,
    )(x1, degp, degp)


def _tc_b(s1, degp, self1, b1):
    return pl.pallas_call(
        _tc_b_body,
        grid=_GRID,
        in_specs=[_slab(NH, 0), _slab(NH, 1), _slab(DEGW, 0), _slab(DEGW, 1),
                  _rows(NH), _whole(1, NH)],
        out_specs=[_rows(NH), _rows(NH)],
        out_shape=[jax.ShapeDtypeStruct((N, NH), _F32)] * 2,
    )(s1, s1, degp, degp, self1, b1)


def _tc_c(s2, degp, self2, w2, b2):
    return pl.pallas_call(
        _tc_c_body,
        grid=_GRID,
        in_specs=[_slab(NH, 0), _slab(NH, 1), _slab(DEGW, 0), _slab(DEGW, 1),
                  _rows(NH), _whole(NH, NL), _whole(1, NL)],
        out_specs=_rows(NL),
        out_shape=jax.ShapeDtypeStruct((N, NL), _F32),
    )(s2, s2, degp, degp, self2, w2, b2)


def kernel(logits, edge_index, W1, b1, W2, b2):
    assert logits.shape == (N, NL) and edge_index.shape == (2, E)
    src = edge_index[0].astype(jnp.int32)
    dst = edge_index[1].astype(jnp.int32)
    pad = EPAD - E
    pad_dst = N + jnp.arange(pad, dtype=jnp.int32) % (ACC_ROWS - N)
    pad_src = jnp.arange(pad, dtype=jnp.int32) % N
    src_p = jnp.concatenate([src, pad_src]).reshape(NPOOL, CH)
    dst_p = jnp.concatenate([dst, pad_dst]).reshape(NPOOL, CH)

    ones16 = jnp.ones((CH, DEGW), _F32)
    zeros16 = jnp.zeros((CH, DEGW), _F32)
    zeros64 = jnp.zeros((CH, NH), _F32)

    degp = _deg_kernel(dst_p, ones16, zeros16)
    x1 = _tc_mm(logits, W1)
    xs, self1 = _tc_scale(x1, degp)
    s1 = _scatter_kernel(xs, src_p, dst_p, zeros64)
    hs, self2 = _tc_b(s1, degp, self1, b1.reshape(1, NH))
    s2 = _scatter_kernel(hs, src_p, dst_p, zeros64)
    return _tc_c(s2, degp, self2, W2, b2.reshape(1, NL))


# speedup vs baseline: 2.0257x; 1.0050x over previous
import functools

import jax
import jax.numpy as jnp
from jax import lax
from jax.experimental import pallas as pl
from jax.experimental.pallas import tpu as pltpu
from jax.experimental.pallas import tpu_sc as plsc

N = 10000
E = 320000
NL = 128
NH = 64
NC = 2
NS = 16
NW = NC * NS
CH = 128
NCH = 79
EPAD = NW * NCH * CH
NPOOL = NW * NCH
FAST_CORE = 1
NCH_FAST = 79
NCH_SLOW = 2 * NCH - NCH_FAST
ACC_ROWS = 10240
DEGW = 16
BM = 2000
_F32 = jnp.float32


def _sc_mesh():
    return plsc.VectorSubcoreMesh(core_axis_name="c", subcore_axis_name="s")


_SC_PARAMS = pltpu.CompilerParams(use_tc_tiling_on_sc=False)


@functools.partial(
    pl.kernel,
    out_type=jax.ShapeDtypeStruct((NC, ACC_ROWS, DEGW), _F32),
    mesh=_sc_mesh(),
    scratch_types=[
        pltpu.VMEM((NCH, CH), jnp.int32),
        pltpu.VMEM((CH, DEGW), _F32),
        pltpu.VMEM((CH, DEGW), _F32),
        pltpu.VMEM((CH, DEGW), _F32),
        pltpu.VMEM_SHARED((ACC_ROWS, DEGW), _F32),
        pltpu.SemaphoreType.DMA,
    ],
    compiler_params=_SC_PARAMS,
)
def _deg_kernel(dst_hbm, ones_hbm, zeros_hbm, out_hbm,
                dst_vm, ones_vm, zero_vm, buf_vm, acc, dsem):
    c = lax.axis_index("c")
    s = lax.axis_index("s")
    wid = c * NS + s
    pltpu.sync_copy(dst_hbm.at[pl.ds(wid * NCH, NCH)], dst_vm)
    pltpu.sync_copy(ones_hbm, ones_vm)
    pltpu.sync_copy(zeros_hbm, zero_vm)
    for z in range(ACC_ROWS // (NS * CH)):
        pltpu.sync_copy(zero_vm, acc.at[pl.ds(s * (ACC_ROWS // NS) + z * CH, CH)])
    plsc.subcore_barrier()

    def dscat(j):
        return pltpu.make_async_copy(ones_vm, acc.at[dst_vm.at[j]], dsem)

    for g in range(0, NCH - 7, 8):
        for j in range(g, g + 8):
            dscat(j).start(add=True)
        for j in range(g, g + 8):
            dscat(j).wait()
    for j in range(NCH - NCH % 8, NCH):
        dscat(j).start(add=True)
    for j in range(NCH - NCH % 8, NCH):
        dscat(j).wait()
    plsc.subcore_barrier()
    for z in range(ACC_ROWS // (NS * CH)):
        r = s * (ACC_ROWS // NS) + z * CH
        pltpu.sync_copy(acc.at[pl.ds(r, CH)], buf_vm)
        pltpu.sync_copy(buf_vm, out_hbm.at[c].at[pl.ds(r, CH)])


@functools.partial(
    pl.kernel,
    out_type=jax.ShapeDtypeStruct((NC, ACC_ROWS, NH), _F32),
    mesh=_sc_mesh(),
    scratch_types=[
        pltpu.VMEM((NCH_FAST, CH), jnp.int32),
        pltpu.VMEM((NCH_FAST, CH), jnp.int32),
        pltpu.VMEM((6, CH, NH), _F32),
        pltpu.VMEM((CH, NH), _F32),
        pltpu.VMEM_SHARED((ACC_ROWS, NH), _F32),
        pltpu.SemaphoreType.DMA((6,)),
        pltpu.SemaphoreType.DMA((6,)),
    ],
    compiler_params=_SC_PARAMS,
)
def _scatter_kernel(y_hbm, src_hbm, dst_hbm, zeros_hbm, out_hbm,
                    src_vm, dst_vm, rows_vm, zero_vm, acc, gsem, ssem):
    c = lax.axis_index("c")
    s = lax.axis_index("s")
    cnt = jnp.where(c == FAST_CORE, NCH_FAST, NCH_SLOW)
    base = jnp.where(c == FAST_CORE, NS * NCH_SLOW + s * NCH_FAST,
                     s * NCH_SLOW)
    pltpu.sync_copy(src_hbm.at[pl.ds(base, NCH_FAST)], src_vm)
    pltpu.sync_copy(dst_hbm.at[pl.ds(base, NCH_FAST)], dst_vm)
    pltpu.sync_copy(zeros_hbm, zero_vm)
    for z in range(ACC_ROWS // (NS * CH)):
        pltpu.sync_copy(zero_vm, acc.at[pl.ds(s * (ACC_ROWS // NS) + z * CH, CH)])
    plsc.subcore_barrier()

    def _gather(j):
        b = lax.rem(j, 6)
        pltpu.async_copy(y_hbm.at[src_vm.at[j]], rows_vm.at[b], gsem.at[b])

    def _scatter(j):
        b = lax.rem(j, 6)
        return pltpu.make_async_copy(rows_vm.at[b], acc.at[dst_vm.at[j]],
                                     ssem.at[b])

    for j in range(5):
        _gather(j)

    def body(j, carry):
        b = lax.rem(j, 6)
        pltpu.make_async_copy(y_hbm.at[src_vm.at[j]], rows_vm.at[b],
                              gsem.at[b]).wait()
        _scatter(j).start(add=True)

        @pl.when(j >= 1)
        def _():
            _scatter(j - 1).wait()

        @pl.when(j + 5 < cnt)
        def _():
            _gather(j + 5)

        return carry

    lax.fori_loop(0, cnt, body, 0)

    def drain(j, carry):
        _scatter(j).wait()
        return carry

    lax.fori_loop(cnt - 1, cnt, drain, 0)
    plsc.subcore_barrier()
    for z in range(ACC_ROWS // (NS * CH)):
        r = s * (ACC_ROWS // NS) + z * CH
        pltpu.sync_copy(acc.at[pl.ds(r, CH)], zero_vm)
        pltpu.sync_copy(zero_vm, out_hbm.at[c].at[pl.ds(r, CH)])


def _deg_stats(da_ref, db_ref):
    deg = da_ref[0, :, 0:1] + db_ref[0, :, 0:1] + 1.0
    return lax.rsqrt(deg), 1.0 / deg


def _tc_mm_body(lg_ref, w_ref, x1_ref):
    x1_ref[...] = jnp.dot(lg_ref[...], w_ref[...], preferred_element_type=_F32)


def _tc_scale_body(x1_ref, da_ref, db_ref, xs_ref, self1_ref):
    dis, inv = _deg_stats(da_ref, db_ref)
    x1 = x1_ref[...]
    xs_ref[...] = x1 * dis
    self1_ref[...] = x1 * inv


def _tc_b_body(sa_ref, sb_ref, da_ref, db_ref, self1_ref, b1_ref,
               hs_ref, self2_ref):
    dis, inv = _deg_stats(da_ref, db_ref)
    h = dis * (sa_ref[0] + sb_ref[0]) + self1_ref[...] + b1_ref[...]
    h = jnp.maximum(h, 0.0)
    hs_ref[...] = h * dis
    self2_ref[...] = h * inv


def _tc_c_body(sa_ref, sb_ref, da_ref, db_ref, self2_ref, w_ref, b2_ref,
               out_ref):
    dis, inv = _deg_stats(da_ref, db_ref)
    p = dis * (sa_ref[0] + sb_ref[0]) + self2_ref[...]
    out_ref[...] = jnp.dot(p, w_ref[...], preferred_element_type=_F32) + b2_ref[...]


def _rows(shape_minor):
    return pl.BlockSpec((BM, shape_minor), lambda i: (i, 0))


def _slab(shape_minor, k):
    return pl.BlockSpec((1, BM, shape_minor), lambda i, _k=k: (_k, i, 0))


def _whole(r, c_):
    return pl.BlockSpec((r, c_), lambda i: (0, 0))


_GRID = (N // BM,)


def _tc_mm(logits, w1):
    return pl.pallas_call(
        _tc_mm_body,
        grid=_GRID,
        in_specs=[_rows(NL), _whole(NL, NH)],
        out_specs=_rows(NH),
        out_shape=jax.ShapeDtypeStruct((N, NH), _F32),
    )(logits, w1)


def _tc_scale(x1, degp):
    return pl.pallas_call(
        _tc_scale_body,
        grid=_GRID,
        in_specs=[_rows(NH), _slab(DEGW, 0), _slab(DEGW, 1)],
        out_specs=[_rows(NH), _rows(NH)],
        out_shape=[jax.ShapeDtypeStruct((N, NH), _F32)] * 2,
    )(x1, degp, degp)


def _tc_b(s1, degp, self1, b1):
    return pl.pallas_call(
        _tc_b_body,
        grid=_GRID,
        in_specs=[_slab(NH, 0), _slab(NH, 1), _slab(DEGW, 0), _slab(DEGW, 1),
                  _rows(NH), _whole(1, NH)],
        out_specs=[_rows(NH), _rows(NH)],
        out_shape=[jax.ShapeDtypeStruct((N, NH), _F32)] * 2,
    )(s1, s1, degp, degp, self1, b1)


def _tc_c(s2, degp, self2, w2, b2):
    return pl.pallas_call(
        _tc_c_body,
        grid=_GRID,
        in_specs=[_slab(NH, 0), _slab(NH, 1), _slab(DEGW, 0), _slab(DEGW, 1),
                  _rows(NH), _whole(NH, NL), _whole(1, NL)],
        out_specs=_rows(NL),
        out_shape=jax.ShapeDtypeStruct((N, NL), _F32),
    )(s2, s2, degp, degp, self2, w2, b2)


def kernel(logits, edge_index, W1, b1, W2, b2):
    assert logits.shape == (N, NL) and edge_index.shape == (2, E)
    src = edge_index[0].astype(jnp.int32)
    dst = edge_index[1].astype(jnp.int32)
    pad = EPAD - E
    pad_dst = N + jnp.arange(pad, dtype=jnp.int32) % (ACC_ROWS - N)
    pad_src = jnp.arange(pad, dtype=jnp.int32) % N
    src_p = jnp.concatenate([src, pad_src]).reshape(NPOOL, CH)
    dst_p = jnp.concatenate([dst, pad_dst]).reshape(NPOOL, CH)

    ones16 = jnp.ones((CH, DEGW), _F32)
    zeros16 = jnp.zeros((CH, DEGW), _F32)
    zeros64 = jnp.zeros((CH, NH), _F32)

    degp = _deg_kernel(dst_p, ones16, zeros16)
    x1 = _tc_mm(logits, W1)
    xs, self1 = _tc_scale(x1, degp)
    s1 = _scatter_kernel(xs, src_p, dst_p, zeros64)
    hs, self2 = _tc_b(s1, degp, self1, b1.reshape(1, NH))
    s2 = _scatter_kernel(hs, src_p, dst_p, zeros64)
    return _tc_c(s2, degp, self2, W2, b2.reshape(1, NL))
